# baseline (speedup 1.0000x reference)
import functools

import jax
import jax.numpy as jnp
from jax import lax
from jax.experimental import pallas as pl
from jax.experimental.pallas import tpu as pltpu
from jax.experimental.pallas import tpu_sc as plsc

N = 10000
E = 320000
D = 128
H = 64
G = 16
C = 6

NC = 2
NS = 16
NW = NC * NS
CHUNK = 128
K = 80
EPAD = NW * CHUNK * K
KA = 88
KB = 72
KS = 2 * KA
NACC = 10048
RPT = NACC // NS
DD = 16


def _sc_agg(dw):
  mesh = plsc.VectorSubcoreMesh(core_axis_name="c", subcore_axis_name="s")

  @functools.partial(
      pl.kernel,
      out_type=jax.ShapeDtypeStruct((NC, NACC, dw), jnp.float32),
      mesh=mesh,
      compiler_params=pltpu.CompilerParams(use_tc_tiling_on_sc=False),
      scratch_types=[
          pltpu.VMEM((KA, CHUNK), jnp.int32),
          pltpu.VMEM((KA, CHUNK), jnp.int32),
          pltpu.VMEM((CHUNK, dw), jnp.float32),
          pltpu.VMEM((CHUNK, dw), jnp.float32),
          pltpu.VMEM_SHARED((NACC, dw), jnp.float32),
          pltpu.VMEM_SHARED((N, dw), jnp.float32),
          pltpu.SemaphoreType.DMA,
      ],
  )
  def agg(y_hbm, src_hbm, dst_hbm, zeros_hbm, out_hbm,
          src_v, dst_v, rows_a, rows_b, acc_sh, y_sh, gsem):
    c = lax.axis_index("c")
    s = lax.axis_index("s")
    base = s * RPT
    pltpu.sync_copy(zeros_hbm, acc_sh.at[pl.ds(base, RPT)])

    @pl.when(c != 0)
    def _():
      pltpu.sync_copy(y_hbm.at[pl.ds(s * (N // NS), N // NS)],
                      y_sh.at[pl.ds(s * (N // NS), N // NS)])

    pltpu.sync_copy(src_hbm.at[s, pl.ds(c * KA, KA)], src_v)
    pltpu.sync_copy(dst_hbm.at[s, pl.ds(c * KA, KA)], dst_v)
    plsc.subcore_barrier()

    def step(jj, carry):
      j0 = 2 * jj
      j1 = 2 * jj + 1

      @pl.when(c == 0)
      def _():
        pltpu.async_copy(y_hbm.at[src_v.at[j0]], rows_a, gsem)
        pltpu.async_copy(y_hbm.at[src_v.at[j1]], rows_b, gsem)

      @pl.when(c != 0)
      def _():
        pltpu.async_copy(y_sh.at[src_v.at[j0]], rows_a, gsem)
        pltpu.async_copy(y_sh.at[src_v.at[j1]], rows_b, gsem)

      pltpu.make_async_copy(y_sh.at[src_v.at[j0]], rows_a, gsem).wait()
      pltpu.sync_copy(rows_a, acc_sh.at[dst_v.at[j0]], add=True)
      pltpu.make_async_copy(y_sh.at[src_v.at[j1]], rows_b, gsem).wait()
      pltpu.sync_copy(rows_b, acc_sh.at[dst_v.at[j1]], add=True)
      return carry

    npairs = jnp.where(c == 0, KA // 2, KB // 2)
    lax.fori_loop(0, npairs, step, 0)
    plsc.subcore_barrier()
    pltpu.sync_copy(acc_sh.at[pl.ds(base, RPT)],
                    out_hbm.at[c, pl.ds(base, RPT)])

  return agg


def _sc_deg():
  mesh = plsc.VectorSubcoreMesh(core_axis_name="c", subcore_axis_name="s")

  @functools.partial(
      pl.kernel,
      out_type=jax.ShapeDtypeStruct((NC, NACC, DD), jnp.float32),
      mesh=mesh,
      compiler_params=pltpu.CompilerParams(use_tc_tiling_on_sc=False),
      scratch_types=[
          pltpu.VMEM((KA, CHUNK), jnp.int32),
          pltpu.VMEM((CHUNK, DD), jnp.float32),
          pltpu.VMEM_SHARED((NACC, DD), jnp.float32),
      ],
  )
  def deg(dst_hbm, zeros_hbm, ones_hbm, out_hbm, dst_v, ones_v, acc_sh):
    c = lax.axis_index("c")
    s = lax.axis_index("s")
    base = s * RPT
    pltpu.sync_copy(zeros_hbm, acc_sh.at[pl.ds(base, RPT)])
    pltpu.sync_copy(ones_hbm, ones_v)
    pltpu.sync_copy(dst_hbm.at[s, pl.ds(c * KA, KA)], dst_v)
    plsc.subcore_barrier()

    def step(j, carry):
      pltpu.sync_copy(ones_v, acc_sh.at[dst_v.at[j]], add=True)
      return carry

    lax.fori_loop(0, jnp.where(c == 0, KA, KB), step, 0)
    plsc.subcore_barrier()
    pltpu.sync_copy(acc_sh.at[pl.ds(base, RPT)],
                    out_hbm.at[c, pl.ds(base, RPT)])

  return deg


def _bn_fold(Wl, Wr, bl, g, be, rm, rv):
  s = g * lax.rsqrt(rv + 1e-5)
  return Wl * s, Wr * s, (bl - rm) * s + be


def _k0_body(x_ref, Wl_ref, Wr_ref, bl_ref, g_ref, be_ref, rm_ref, rv_ref,
             y_ref, z_ref):
  Wls, Wrs, b = _bn_fold(Wl_ref[...], Wr_ref[...], bl_ref[...], g_ref[...],
                         be_ref[...], rm_ref[...], rv_ref[...])
  x = x_ref[...]
  y_ref[...] = jnp.dot(x, Wls, preferred_element_type=jnp.float32)
  z_ref[...] = jnp.dot(x, Wrs, preferred_element_type=jnp.float32) + b


def _k1_body(acc_ref, deg_ref, z_ref, Wl_ref, Wr_ref, bl_ref, g_ref, be_ref,
             rm_ref, rv_ref, y_ref, zo_ref, inv_ref):
  accsum = acc_ref[0, :N, :] + acc_ref[1, :N, :]
  deg = deg_ref[0, :N, 0:1] + deg_ref[1, :N, 0:1]
  inv = 1.0 / jnp.maximum(deg, 1.0)
  h = jnp.maximum(accsum * inv + z_ref[...], 0.0)
  Wls, Wrs, b = _bn_fold(Wl_ref[...], Wr_ref[...], bl_ref[...], g_ref[...],
                         be_ref[...], rm_ref[...], rv_ref[...])
  y_ref[...] = jnp.dot(h, Wls, preferred_element_type=jnp.float32)
  zo_ref[...] = jnp.dot(h, Wrs, preferred_element_type=jnp.float32) + b
  inv_ref[...] = inv


def _k2_body(acc_ref, z_ref, inv_ref, Wl_ref, Wr_ref, bl_ref, g_ref, be_ref,
             rm_ref, rv_ref, y_ref, zo_ref):
  accsum = acc_ref[0, :N, :] + acc_ref[1, :N, :]
  h = jnp.maximum(accsum * inv_ref[...] + z_ref[...], 0.0)
  Wls, Wrs, b = _bn_fold(Wl_ref[...], Wr_ref[...], bl_ref[...], g_ref[...],
                         be_ref[...], rm_ref[...], rv_ref[...])
  y_ref[...] = jnp.dot(h, Wls, preferred_element_type=jnp.float32)
  zo_ref[...] = jnp.dot(h, Wrs, preferred_element_type=jnp.float32) + b


def _k3_body(acc_ref, z_ref, inv_ref, batch_ref, hW1_ref, hb1_ref, hW2_ref,
             hb2_ref, out_ref):
  accsum = acc_ref[0, :N, :] + acc_ref[1, :N, :]
  h = jnp.maximum(accsum * inv_ref[...] + z_ref[...], 0.0)
  cols = lax.broadcasted_iota(jnp.int32, (N, G), 1)
  oh = (batch_ref[...] == cols).astype(jnp.float32)
  hx = jnp.concatenate([h, jnp.ones((N, 1), jnp.float32)], axis=1)
  px = lax.dot_general(oh, hx, (((0,), (0,)), ((), ())),
                       preferred_element_type=jnp.float32)
  pooled = px[:, :H] / jnp.maximum(px[:, H:H + 1], 1.0)
  hh = jnp.maximum(
      jnp.dot(pooled, hW1_ref[...], preferred_element_type=jnp.float32)
      + hb1_ref[...], 0.0)
  out_ref[...] = (jnp.dot(hh, hW2_ref[...], preferred_element_type=jnp.float32)
                  + hb2_ref[...])


def kernel(x, edge_index, batch, Wl0, bl0, Wr0, g0, be0, rm0, rv0,
           Wl1, bl1, Wr1, g1, be1, rm1, rv1, Wl2, bl2, Wr2, g2, be2, rm2, rv2,
           hW1, hb1, hW2, hb2):
  f32 = jnp.float32
  src = jnp.pad(
      jnp.concatenate([edge_index[0], jnp.zeros((EPAD - E,), jnp.int32)])
      .reshape(NS, 2 * K, CHUNK), ((0, 0), (0, KS - 2 * K), (0, 0)))
  dst = jnp.pad(
      jnp.concatenate([edge_index[1], jnp.full((EPAD - E,), N, jnp.int32)])
      .reshape(NS, 2 * K, CHUNK), ((0, 0), (0, KS - 2 * K), (0, 0)),
      constant_values=N)
  zeros16 = jnp.zeros((RPT, DD), f32)
  ones16 = jnp.ones((CHUNK, DD), f32)
  zeros64 = jnp.zeros((RPT, H), f32)
  batch2 = batch.reshape(N, 1)
  r = lambda v: v.reshape(1, -1)
  hW2p = jnp.zeros((H // 2, 128), f32).at[:, :C].set(hW2)
  hb2p = jnp.zeros((1, 128), f32).at[0, :C].set(hb2)

  dega = _sc_deg()(dst, zeros16, ones16)

  y0, z0 = pl.pallas_call(
      _k0_body,
      out_shape=[jax.ShapeDtypeStruct((N, H), f32),
                 jax.ShapeDtypeStruct((N, H), f32)],
  )(x, Wl0, Wr0, r(bl0), r(g0), r(be0), r(rm0), r(rv0))

  acc0 = _sc_agg(H)(y0, src, dst, zeros64)

  y1, z1, inv = pl.pallas_call(
      _k1_body,
      out_shape=[jax.ShapeDtypeStruct((N, H), f32),
                 jax.ShapeDtypeStruct((N, H), f32),
                 jax.ShapeDtypeStruct((N, 1), f32)],
  )(acc0, dega, z0, Wl1, Wr1, r(bl1), r(g1), r(be1), r(rm1), r(rv1))

  acc1 = _sc_agg(H)(y1, src, dst, zeros64)

  y2, z2 = pl.pallas_call(
      _k2_body,
      out_shape=[jax.ShapeDtypeStruct((N, H), f32),
                 jax.ShapeDtypeStruct((N, H), f32)],
  )(acc1, z1, inv, Wl2, Wr2, r(bl2), r(g2), r(be2), r(rm2), r(rv2))

  acc2 = _sc_agg(H)(y2, src, dst, zeros64)

  out = pl.pallas_call(
      _k3_body,
      out_shape=jax.ShapeDtypeStruct((G, 128), f32),
  )(acc2, z2, inv, batch2, hW1, r(hb1), hW2p, hb2p)
  return out[:, :C]


# speedup vs baseline: 1.0333x; 1.0333x over previous
import functools

import jax
import jax.numpy as jnp
from jax import lax
from jax.experimental import pallas as pl
from jax.experimental.pallas import tpu as pltpu
from jax.experimental.pallas import tpu_sc as plsc

N = 10000
E = 320000
D = 128
H = 64
G = 16
C = 6

NC = 2
NS = 16
NW = NC * NS
CHUNK = 128
K = 80
EPAD = NW * CHUNK * K
KA = 92
KB = 68
KS = 2 * KA
NACC = 10048
RPT = NACC // NS
DD = 16


def _sc_agg(dw):
  mesh = plsc.VectorSubcoreMesh(core_axis_name="c", subcore_axis_name="s")

  @functools.partial(
      pl.kernel,
      out_type=jax.ShapeDtypeStruct((NC, NACC, dw), jnp.float32),
      mesh=mesh,
      compiler_params=pltpu.CompilerParams(use_tc_tiling_on_sc=False),
      scratch_types=[
          pltpu.VMEM((KA, CHUNK), jnp.int32),
          pltpu.VMEM((KA, CHUNK), jnp.int32),
          pltpu.VMEM((CHUNK, dw), jnp.float32),
          pltpu.VMEM((CHUNK, dw), jnp.float32),
          pltpu.VMEM_SHARED((NACC, dw), jnp.float32),
          pltpu.VMEM_SHARED((N, dw), jnp.float32),
          pltpu.SemaphoreType.DMA,
      ],
  )
  def agg(y_hbm, src_hbm, dst_hbm, zeros_hbm, out_hbm,
          src_v, dst_v, rows_a, rows_b, acc_sh, y_sh, gsem):
    c = lax.axis_index("c")
    s = lax.axis_index("s")
    base = s * RPT
    pltpu.sync_copy(zeros_hbm, acc_sh.at[pl.ds(base, RPT)])
    pltpu.sync_copy(y_hbm.at[pl.ds(s * (N // NS), N // NS)],
                    y_sh.at[pl.ds(s * (N // NS), N // NS)])
    pltpu.sync_copy(src_hbm.at[s, pl.ds(c * KA, KA)], src_v)
    pltpu.sync_copy(dst_hbm.at[s, pl.ds(c * KA, KA)], dst_v)
    plsc.subcore_barrier()

    def step(jj, carry):
      j0 = 2 * jj
      j1 = 2 * jj + 1
      d0 = pltpu.async_copy(y_sh.at[src_v.at[j0]], rows_a, gsem)
      d1 = pltpu.async_copy(y_sh.at[src_v.at[j1]], rows_b, gsem)
      d0.wait()
      pltpu.sync_copy(rows_a, acc_sh.at[dst_v.at[j0]], add=True)
      d1.wait()
      pltpu.sync_copy(rows_b, acc_sh.at[dst_v.at[j1]], add=True)
      return carry

    npairs = jnp.where(c == 0, KA // 2, KB // 2)
    lax.fori_loop(0, npairs, step, 0)
    plsc.subcore_barrier()
    pltpu.sync_copy(acc_sh.at[pl.ds(base, RPT)],
                    out_hbm.at[c, pl.ds(base, RPT)])

  return agg


def _sc_deg():
  mesh = plsc.VectorSubcoreMesh(core_axis_name="c", subcore_axis_name="s")

  @functools.partial(
      pl.kernel,
      out_type=jax.ShapeDtypeStruct((NC, NACC, DD), jnp.float32),
      mesh=mesh,
      compiler_params=pltpu.CompilerParams(use_tc_tiling_on_sc=False),
      scratch_types=[
          pltpu.VMEM((KA, CHUNK), jnp.int32),
          pltpu.VMEM((CHUNK, DD), jnp.float32),
          pltpu.VMEM_SHARED((NACC, DD), jnp.float32),
      ],
  )
  def deg(dst_hbm, zeros_hbm, ones_hbm, out_hbm, dst_v, ones_v, acc_sh):
    c = lax.axis_index("c")
    s = lax.axis_index("s")
    base = s * RPT
    pltpu.sync_copy(zeros_hbm, acc_sh.at[pl.ds(base, RPT)])
    pltpu.sync_copy(ones_hbm, ones_v)
    pltpu.sync_copy(dst_hbm.at[s, pl.ds(c * KA, KA)], dst_v)
    plsc.subcore_barrier()

    def step(j, carry):
      pltpu.sync_copy(ones_v, acc_sh.at[dst_v.at[j]], add=True)
      return carry

    lax.fori_loop(0, jnp.where(c == 0, KA, KB), step, 0)
    plsc.subcore_barrier()
    pltpu.sync_copy(acc_sh.at[pl.ds(base, RPT)],
                    out_hbm.at[c, pl.ds(base, RPT)])

  return deg


def _bn_fold(Wl, Wr, bl, g, be, rm, rv):
  s = g * lax.rsqrt(rv + 1e-5)
  return Wl * s, Wr * s, (bl - rm) * s + be


def _k0_body(x_ref, Wl_ref, Wr_ref, bl_ref, g_ref, be_ref, rm_ref, rv_ref,
             y_ref, z_ref):
  Wls, Wrs, b = _bn_fold(Wl_ref[...], Wr_ref[...], bl_ref[...], g_ref[...],
                         be_ref[...], rm_ref[...], rv_ref[...])
  x = x_ref[...]
  y_ref[...] = jnp.dot(x, Wls, preferred_element_type=jnp.float32)
  z_ref[...] = jnp.dot(x, Wrs, preferred_element_type=jnp.float32) + b


def _k1_body(acc_ref, deg_ref, z_ref, Wl_ref, Wr_ref, bl_ref, g_ref, be_ref,
             rm_ref, rv_ref, y_ref, zo_ref, inv_ref):
  accsum = acc_ref[0, :N, :] + acc_ref[1, :N, :]
  deg = deg_ref[0, :N, 0:1] + deg_ref[1, :N, 0:1]
  inv = 1.0 / jnp.maximum(deg, 1.0)
  h = jnp.maximum(accsum * inv + z_ref[...], 0.0)
  Wls, Wrs, b = _bn_fold(Wl_ref[...], Wr_ref[...], bl_ref[...], g_ref[...],
                         be_ref[...], rm_ref[...], rv_ref[...])
  y_ref[...] = jnp.dot(h, Wls, preferred_element_type=jnp.float32)
  zo_ref[...] = jnp.dot(h, Wrs, preferred_element_type=jnp.float32) + b
  inv_ref[...] = inv


def _k2_body(acc_ref, z_ref, inv_ref, Wl_ref, Wr_ref, bl_ref, g_ref, be_ref,
             rm_ref, rv_ref, y_ref, zo_ref):
  accsum = acc_ref[0, :N, :] + acc_ref[1, :N, :]
  h = jnp.maximum(accsum * inv_ref[...] + z_ref[...], 0.0)
  Wls, Wrs, b = _bn_fold(Wl_ref[...], Wr_ref[...], bl_ref[...], g_ref[...],
                         be_ref[...], rm_ref[...], rv_ref[...])
  y_ref[...] = jnp.dot(h, Wls, preferred_element_type=jnp.float32)
  zo_ref[...] = jnp.dot(h, Wrs, preferred_element_type=jnp.float32) + b


def _k3_body(acc_ref, z_ref, inv_ref, batch_ref, hW1_ref, hb1_ref, hW2_ref,
             hb2_ref, out_ref):
  accsum = acc_ref[0, :N, :] + acc_ref[1, :N, :]
  h = jnp.maximum(accsum * inv_ref[...] + z_ref[...], 0.0)
  cols = lax.broadcasted_iota(jnp.int32, (N, G), 1)
  oh = (batch_ref[...] == cols).astype(jnp.float32)
  hx = jnp.concatenate([h, jnp.ones((N, 1), jnp.float32)], axis=1)
  px = lax.dot_general(oh, hx, (((0,), (0,)), ((), ())),
                       preferred_element_type=jnp.float32)
  pooled = px[:, :H] / jnp.maximum(px[:, H:H + 1], 1.0)
  hh = jnp.maximum(
      jnp.dot(pooled, hW1_ref[...], preferred_element_type=jnp.float32)
      + hb1_ref[...], 0.0)
  out_ref[...] = (jnp.dot(hh, hW2_ref[...], preferred_element_type=jnp.float32)
                  + hb2_ref[...])


def kernel(x, edge_index, batch, Wl0, bl0, Wr0, g0, be0, rm0, rv0,
           Wl1, bl1, Wr1, g1, be1, rm1, rv1, Wl2, bl2, Wr2, g2, be2, rm2, rv2,
           hW1, hb1, hW2, hb2):
  f32 = jnp.float32
  src = jnp.pad(
      jnp.concatenate([edge_index[0], jnp.zeros((EPAD - E,), jnp.int32)])
      .reshape(NS, 2 * K, CHUNK), ((0, 0), (0, KS - 2 * K), (0, 0)))
  dst = jnp.pad(
      jnp.concatenate([edge_index[1], jnp.full((EPAD - E,), N, jnp.int32)])
      .reshape(NS, 2 * K, CHUNK), ((0, 0), (0, KS - 2 * K), (0, 0)),
      constant_values=N)
  zeros16 = jnp.zeros((RPT, DD), f32)
  ones16 = jnp.ones((CHUNK, DD), f32)
  zeros64 = jnp.zeros((RPT, H), f32)
  batch2 = batch.reshape(N, 1)
  r = lambda v: v.reshape(1, -1)
  hW2p = jnp.zeros((H // 2, 128), f32).at[:, :C].set(hW2)
  hb2p = jnp.zeros((1, 128), f32).at[0, :C].set(hb2)

  dega = _sc_deg()(dst, zeros16, ones16)

  y0, z0 = pl.pallas_call(
      _k0_body,
      out_shape=[jax.ShapeDtypeStruct((N, H), f32),
                 jax.ShapeDtypeStruct((N, H), f32)],
  )(x, Wl0, Wr0, r(bl0), r(g0), r(be0), r(rm0), r(rv0))

  acc0 = _sc_agg(H)(y0, src, dst, zeros64)

  y1, z1, inv = pl.pallas_call(
      _k1_body,
      out_shape=[jax.ShapeDtypeStruct((N, H), f32),
                 jax.ShapeDtypeStruct((N, H), f32),
                 jax.ShapeDtypeStruct((N, 1), f32)],
  )(acc0, dega, z0, Wl1, Wr1, r(bl1), r(g1), r(be1), r(rm1), r(rv1))

  acc1 = _sc_agg(H)(y1, src, dst, zeros64)

  y2, z2 = pl.pallas_call(
      _k2_body,
      out_shape=[jax.ShapeDtypeStruct((N, H), f32),
                 jax.ShapeDtypeStruct((N, H), f32)],
  )(acc1, z1, inv, Wl2, Wr2, r(bl2), r(g2), r(be2), r(rm2), r(rv2))

  acc2 = _sc_agg(H)(y2, src, dst, zeros64)

  out = pl.pallas_call(
      _k3_body,
      out_shape=jax.ShapeDtypeStruct((G, 128), f32),
  )(acc2, z2, inv, batch2, hW1, r(hb1), hW2p, hb2p)
  return out[:, :C]
